# Initial kernel scaffold; baseline (speedup 1.0000x reference)
import jax
import jax.numpy as jnp
from jax.experimental import pallas as pl


def kernel(z, pos, edge_index, batch, params):
    raise NotImplementedError("write your pallas kernel here")


# speedup vs baseline: 2.9427x; 2.9427x over previous
import dataclasses
import functools

import jax
import jax.numpy as jnp
from jax import lax
from jax.experimental import pallas as pl
from jax.experimental.pallas import tpu as pltpu
from jax.experimental.pallas import tpu_sc as plsc

N = 10000
E = 320000
H = 128
NUM_GRAPHS = 256

NC, NS = 2, 16
NW = NC * NS
EPW = E // NW
CH = 80
NCH = EPW // CH
NG = CH // 16
EPS = E // NS
NCHS = EPS // CH
HALF = N // 2
ACCR = HALF + 8
EPQ = E // 4
SB = 4000
NSB = EPQ // SB
ZR = 16

NBLK = 1000
EBLK = 1000


@functools.cache
def _sc_mesh():
    return plsc.VectorSubcoreMesh(core_axis_name="c", subcore_axis_name="s")


@functools.cache
def _sc_params():
    cp = pltpu.CompilerParams()
    if "needs_layout_passes" in pltpu.CompilerParams.__dataclass_fields__:
        cp = dataclasses.replace(cp, needs_layout_passes=False)
    return cp


@jax.jit
def _sc_gather(tsrc, tdst, src_w, dst_w, px, py, pz):
    @functools.partial(
        pl.kernel,
        out_type=(
            jax.ShapeDtypeStruct((E, H), jnp.float32),
            jax.ShapeDtypeStruct((E, H), jnp.float32),
            jax.ShapeDtypeStruct((E,), jnp.float32),
            jax.ShapeDtypeStruct((E,), jnp.float32),
            jax.ShapeDtypeStruct((E,), jnp.float32),
        ),
        mesh=_sc_mesh(),
        compiler_params=_sc_params(),
        scratch_types=[
            pltpu.VMEM((NCH, CH), jnp.int32),
            pltpu.VMEM((NCH, CH), jnp.int32),
            pltpu.VMEM((CH, H), jnp.float32),
            pltpu.VMEM((CH, H), jnp.float32),
            pltpu.VMEM((N,), jnp.float32),
            pltpu.VMEM((N,), jnp.float32),
            pltpu.VMEM((N,), jnp.float32),
            pltpu.VMEM((EPW,), jnp.float32),
            pltpu.VMEM((EPW,), jnp.float32),
            pltpu.VMEM((EPW,), jnp.float32),
            pltpu.SemaphoreType.DMA,
            pltpu.SemaphoreType.DMA,
        ],
    )
    def k(tsrc_hbm, tdst_hbm, src_hbm, dst_hbm, px_hbm, py_hbm, pz_hbm,
          gs_hbm, gd_hbm, rx_hbm, ry_hbm, rz_hbm,
          isv, idv, bs, bd, pxv, pyv, pzv, rbx, rby, rbz, sem_s, sem_d):
        wid = lax.axis_index("s") * NC + lax.axis_index("c")
        pltpu.sync_copy(src_hbm.at[wid], isv)
        pltpu.sync_copy(dst_hbm.at[wid], idv)
        pltpu.sync_copy(px_hbm, pxv)
        pltpu.sync_copy(py_hbm, pyv)
        pltpu.sync_copy(pz_hbm, pzv)
        base = wid * EPW

        @pl.loop(0, NCH)
        def _(j):
            cs = pltpu.async_copy(tsrc_hbm.at[isv.at[j]], bs, sem_s)
            cd = pltpu.async_copy(tdst_hbm.at[idv.at[j]], bd, sem_d)

            @pl.loop(0, NG)
            def _(g):
                is16 = isv.at[j][pl.ds(g * 16, 16)]
                id16 = idv.at[j][pl.ds(g * 16, 16)]
                o = pl.ds(j * CH + g * 16, 16)
                rbx.at[o][...] = (plsc.load_gather(pxv, [is16])
                                  - plsc.load_gather(pxv, [id16]))
                rby.at[o][...] = (plsc.load_gather(pyv, [is16])
                                  - plsc.load_gather(pyv, [id16]))
                rbz.at[o][...] = (plsc.load_gather(pzv, [is16])
                                  - plsc.load_gather(pzv, [id16]))

            cs.wait()
            cd.wait()
            pltpu.sync_copy(bs, gs_hbm.at[pl.ds(base + j * CH, CH)])
            pltpu.sync_copy(bd, gd_hbm.at[pl.ds(base + j * CH, CH)])

        pltpu.sync_copy(rbx, rx_hbm.at[pl.ds(base, EPW)])
        pltpu.sync_copy(rby, ry_hbm.at[pl.ds(base, EPW)])
        pltpu.sync_copy(rbz, rz_hbm.at[pl.ds(base, EPW)])

    return k(tsrc, tdst, src_w, dst_w, px, py, pz)


@jax.jit
def _sc_scatter(m, rwx, rwy, rwz, dst_s, dst_f):
    @functools.partial(
        pl.kernel,
        out_type=(
            jax.ShapeDtypeStruct((N, H), jnp.float32),
            jax.ShapeDtypeStruct((N // NBLK, 4, 1, NBLK), jnp.float32),
            jax.ShapeDtypeStruct((N // NBLK, 4, 1, NBLK), jnp.float32),
            jax.ShapeDtypeStruct((N // NBLK, 4, 1, NBLK), jnp.float32),
            jax.ShapeDtypeStruct((N // NBLK, 4, 1, NBLK), jnp.float32),
        ),
        mesh=_sc_mesh(),
        compiler_params=_sc_params(),
        scratch_types=[
            pltpu.VMEM((NCHS, CH), jnp.int32),
            pltpu.VMEM((CH, H), jnp.float32),
            pltpu.VMEM((ZR, H), jnp.float32),
            pltpu.VMEM((SB,), jnp.int32),
            pltpu.VMEM((SB,), jnp.float32),
            pltpu.VMEM((N,), jnp.float32),
            pltpu.VMEM((N,), jnp.int32),
            pltpu.VMEM((1, CH), jnp.int32),
            pltpu.VMEM_SHARED((ACCR, H), jnp.float32),
            pltpu.SemaphoreType.DMA,
        ],
    )
    def k(m_hbm, rwx_hbm, rwy_hbm, rwz_hbm, dst_hbm, dstf_hbm,
          agg_hbm, ax_hbm, ay_hbm, az_hbm, ac_hbm,
          idxv, mbuf, zbuf, idxb, vb, acc1, markv, idm1, acc_sh, sem):
        cid = lax.axis_index("c")
        sid = lax.axis_index("s")

        z16 = jnp.zeros((16,), jnp.float32)

        @pl.loop(0, ZR)
        def _(r):
            @pl.loop(0, H // 16)
            def _(c):
                zbuf.at[r, pl.ds(c * 16, 16)][...] = z16

        zlo = sid * 320

        @pl.loop(0, 20)
        def _(r):
            @pl.when((zlo + r * ZR) < ACCR)
            def _():
                pltpu.sync_copy(zbuf, acc_sh.at[pl.ds(zlo + r * ZR, ZR)])

        @pl.loop(0, N // 16)
        def _(r):
            acc1.at[pl.ds(r * 16, 16)][...] = z16

        pltpu.sync_copy(dst_hbm.at[sid], idxv)

        lo = cid * HALF

        @pl.loop(0, NCHS)
        def _(j):
            @pl.loop(0, NG)
            def _(g):
                o = pl.ds(g * 16, 16)
                v = idxv.at[j][o]
                inr = (v >= lo) & (v < lo + HALF)
                idxv.at[j][o] = jnp.where(inr, v - lo, HALF)

        plsc.subcore_barrier()

        base = sid * EPS

        iota16 = lax.iota(jnp.int32, 16)

        @pl.loop(0, NCHS)
        def _(j):
            cm = pltpu.async_copy(
                m_hbm.at[pl.ds(base + j * CH, CH)], mbuf, sem)

            rem0 = []
            for g in range(NG):
                vg = idxv.at[j][pl.ds(g * 16, 16)]
                rem0.append(vg != HALF)
            cm.wait()

            def cond(carry):
                return carry[NG] > 0

            def body(carry):
                rem = carry[:NG]
                slot = []
                for g in range(NG):
                    vg = idxv.at[j][pl.ds(g * 16, 16)]
                    sg = j * 128 + g * 16 + iota16
                    slot.append((vg, sg))
                    plsc.store_scatter(markv, [vg], sg, mask=rem[g])
                newrem = []
                total = jnp.zeros((), jnp.int32)
                for g in range(NG):
                    vg, sg = slot[g]
                    rb = plsc.load_gather(markv, [vg])
                    own = rem[g] & (rb == sg)
                    idm1.at[0][pl.ds(g * 16, 16)] = jnp.where(own, vg, HALF)
                    nr = rem[g] & jnp.logical_not(own)
                    newrem.append(nr)
                    total = total + jnp.sum(nr.astype(jnp.int32))
                pltpu.sync_copy(mbuf, acc_sh.at[idm1.at[0]], add=True)
                return tuple(newrem) + (total,)

            n0 = jnp.zeros((), jnp.int32)
            for g in range(NG):
                n0 = n0 + jnp.sum(rem0[g].astype(jnp.int32))
            lax.while_loop(cond, body, tuple(rem0) + (n0,))

        one16 = jnp.ones((16,), jnp.float32)
        combo = cid * 8 + sid
        comp = combo % 4
        quarter = combo // 4

        @pl.when(sid < 8)
        def _():
            @pl.loop(0, NSB)
            def _(b):
                qbase = quarter * EPQ + b * SB
                pltpu.sync_copy(dstf_hbm.at[pl.ds(qbase, SB)], idxb)

                @pl.when(comp == 0)
                def _():
                    pltpu.sync_copy(rwx_hbm.at[pl.ds(qbase, SB)], vb)

                @pl.when(comp == 1)
                def _():
                    pltpu.sync_copy(rwy_hbm.at[pl.ds(qbase, SB)], vb)

                @pl.when(comp == 2)
                def _():
                    pltpu.sync_copy(rwz_hbm.at[pl.ds(qbase, SB)], vb)

                @pl.loop(0, SB // 16)
                def _(g):
                    o = pl.ds(g * 16, 16)
                    id16 = idxb.at[o][...]

                    @pl.when(comp < 3)
                    def _():
                        plsc.addupdate_scatter(acc1, [id16], vb.at[o][...])

                    @pl.when(comp == 3)
                    def _():
                        plsc.addupdate_scatter(acc1, [id16], one16)

        plsc.subcore_barrier()

        dlo = sid * 320

        @pl.when(sid < NS - 1)
        def _():
            pltpu.sync_copy(acc_sh.at[pl.ds(dlo, 320)],
                            agg_hbm.at[pl.ds(cid * HALF + dlo, 320)])

        @pl.when(sid == NS - 1)
        def _():
            pltpu.sync_copy(acc_sh.at[pl.ds(dlo, 200)],
                            agg_hbm.at[pl.ds(cid * HALF + dlo, 200)])

        @pl.when(sid < 8)
        def _():
            @pl.loop(0, N // NBLK)
            def _(t):
                o = pl.ds(t * NBLK, NBLK)

                @pl.when(comp == 0)
                def _():
                    pltpu.sync_copy(acc1.at[o], ax_hbm.at[t, quarter, 0])

                @pl.when(comp == 1)
                def _():
                    pltpu.sync_copy(acc1.at[o], ay_hbm.at[t, quarter, 0])

                @pl.when(comp == 2)
                def _():
                    pltpu.sync_copy(acc1.at[o], az_hbm.at[t, quarter, 0])

                @pl.when(comp == 3)
                def _():
                    pltpu.sync_copy(acc1.at[o], ac_hbm.at[t, quarter, 0])

    return k(m, rwx, rwy, rwz, dst_s, dst_f)


def _silu(x):
    return x * (1.0 / (1.0 + jnp.exp(-x)))


def _embed_body(z_ref, emb_ref, h_ref):
    zb = z_ref[0, 0, :]
    oh = (zb[:, None] == lax.broadcasted_iota(jnp.int32, (NBLK, H), 1))
    h_ref[...] = jnp.dot(oh.astype(jnp.float32), emb_ref[...],
                         preferred_element_type=jnp.float32)


@jax.jit
def _tc_embed(z3, emb_p):
    return pl.pallas_call(
        _embed_body,
        grid=(N // NBLK,),
        in_specs=[
            pl.BlockSpec((1, 1, NBLK), lambda i: (i, 0, 0)),
            pl.BlockSpec((H, H), lambda i: (0, 0)),
        ],
        out_specs=pl.BlockSpec((NBLK, H), lambda i: (i, 0)),
        out_shape=jax.ShapeDtypeStruct((N, H), jnp.float32),
    )(z3, emb_p)


def _tables_body(h_ref, wa_ref, wb_ref, b1_ref, ts_ref, td_ref):
    h = h_ref[...]
    ts_ref[...] = jnp.dot(h, wa_ref[...], preferred_element_type=jnp.float32)
    td_ref[...] = (jnp.dot(h, wb_ref[...], preferred_element_type=jnp.float32)
                   + b1_ref[...])


@jax.jit
def _tc_tables(h, wa, wb, b1):
    return pl.pallas_call(
        _tables_body,
        grid=(N // NBLK,),
        in_specs=[
            pl.BlockSpec((NBLK, H), lambda i: (i, 0)),
            pl.BlockSpec((H, H), lambda i: (0, 0)),
            pl.BlockSpec((H, H), lambda i: (0, 0)),
            pl.BlockSpec((1, H), lambda i: (0, 0)),
        ],
        out_specs=(
            pl.BlockSpec((NBLK, H), lambda i: (i, 0)),
            pl.BlockSpec((NBLK, H), lambda i: (i, 0)),
        ),
        out_shape=(
            jax.ShapeDtypeStruct((N, H), jnp.float32),
            jax.ShapeDtypeStruct((N, H), jnp.float32),
        ),
    )(h, wa, wb, b1)


def _edge_body(gs_ref, gd_ref, rx_ref, ry_ref, rz_ref,
               wd_ref, w2_ref, b2_ref, wct_ref, cst_ref,
               m_ref, ox_ref, oy_ref, oz_ref):
    rx = rx_ref[0, 0, :]
    ry = ry_ref[0, 0, :]
    rz = rz_ref[0, 0, :]
    dist2 = (rx * rx + ry * ry + rz * rz)[:, None]
    t = gs_ref[...] + gd_ref[...] + dist2 * wd_ref[...]
    m1 = _silu(t)
    m = _silu(jnp.dot(m1, w2_ref[...], preferred_element_type=jnp.float32)
              + b2_ref[...])
    wc = jnp.sum(m * wct_ref[...], axis=1) + cst_ref[0, 0]
    m_ref[...] = m
    ox_ref[0, 0, :] = rx * wc
    oy_ref[0, 0, :] = ry * wc
    oz_ref[0, 0, :] = rz * wc


@jax.jit
def _tc_edge(gs, gd, rx3, ry3, rz3, wd, w2, b2, wct, cst):
    v3 = pl.BlockSpec((1, 1, EBLK), lambda i: (i, 0, 0))
    w128 = pl.BlockSpec((1, H), lambda i: (0, 0))
    return pl.pallas_call(
        _edge_body,
        grid=(E // EBLK,),
        in_specs=[
            pl.BlockSpec((EBLK, H), lambda i: (i, 0)),
            pl.BlockSpec((EBLK, H), lambda i: (i, 0)),
            v3, v3, v3,
            w128,
            pl.BlockSpec((H, H), lambda i: (0, 0)),
            w128, w128,
            pl.BlockSpec((8, 128), lambda i: (0, 0)),
        ],
        out_specs=(
            pl.BlockSpec((EBLK, H), lambda i: (i, 0)),
            v3, v3, v3,
        ),
        out_shape=(
            jax.ShapeDtypeStruct((E, H), jnp.float32),
            jax.ShapeDtypeStruct((E // EBLK, 1, EBLK), jnp.float32),
            jax.ShapeDtypeStruct((E // EBLK, 1, EBLK), jnp.float32),
            jax.ShapeDtypeStruct((E // EBLK, 1, EBLK), jnp.float32),
        ),
    )(gs, gd, rx3, ry3, rz3, wd, w2, b2, wct, cst)


def _node_body(agg_ref, ax_ref, ay_ref, az_ref, ac_ref,
               px_ref, py_ref, pz_ref, h_ref,
               wna_ref, wnb_ref, bn1_ref, wn2_ref, bn2_ref,
               h_out, px_out, py_out, pz_out):
    agg = agg_ref[...]
    deg = jnp.sum(ac_ref[0, :, 0, :], axis=0) + 1.0
    px_out[0, 0, :] = px_ref[0, 0, :] + jnp.sum(ax_ref[0, :, 0, :], axis=0) / deg
    py_out[0, 0, :] = py_ref[0, 0, :] + jnp.sum(ay_ref[0, :, 0, :], axis=0) / deg
    pz_out[0, 0, :] = pz_ref[0, 0, :] + jnp.sum(az_ref[0, :, 0, :], axis=0) / deg
    h = h_ref[...]
    t = _silu(jnp.dot(h, wna_ref[...], preferred_element_type=jnp.float32)
              + jnp.dot(agg, wnb_ref[...], preferred_element_type=jnp.float32)
              + bn1_ref[...])
    h_out[...] = h + jnp.dot(t, wn2_ref[...],
                             preferred_element_type=jnp.float32) + bn2_ref[...]


@jax.jit
def _tc_node(agg, ax, ay, az, ac, px3, py3, pz3, h,
             wna, wnb, bn1, wn2, bn2):
    v3 = pl.BlockSpec((1, 1, NBLK), lambda i: (i, 0, 0))
    part = pl.BlockSpec((1, 4, 1, NBLK), lambda i: (i, 0, 0, 0))
    return pl.pallas_call(
        _node_body,
        grid=(N // NBLK,),
        in_specs=[
            pl.BlockSpec((NBLK, H), lambda i: (i, 0)),
            part, part, part, part,
            v3, v3, v3,
            pl.BlockSpec((NBLK, H), lambda i: (i, 0)),
            pl.BlockSpec((H, H), lambda i: (0, 0)),
            pl.BlockSpec((H, H), lambda i: (0, 0)),
            pl.BlockSpec((1, H), lambda i: (0, 0)),
            pl.BlockSpec((H, H), lambda i: (0, 0)),
            pl.BlockSpec((1, H), lambda i: (0, 0)),
        ],
        out_specs=(
            pl.BlockSpec((NBLK, H), lambda i: (i, 0)),
            v3, v3, v3,
        ),
        out_shape=(
            jax.ShapeDtypeStruct((N, H), jnp.float32),
            jax.ShapeDtypeStruct((N // NBLK, 1, NBLK), jnp.float32),
            jax.ShapeDtypeStruct((N // NBLK, 1, NBLK), jnp.float32),
            jax.ShapeDtypeStruct((N // NBLK, 1, NBLK), jnp.float32),
        ),
    )(agg, ax, ay, az, ac, px3, py3, pz3, h, wna, wnb, bn1, wn2, bn2)


def _pool_body(b_ref, h_ref, wo1_ref, bo1_ref, wo2t_ref, cst_ref, out_ref,
               acc_ref):
    i = pl.program_id(0)

    @pl.when(i == 0)
    def _():
        acc_ref[...] = jnp.zeros((NUM_GRAPHS, H), jnp.float32)

    bb = b_ref[0, 0, :]
    oh = (bb[None, :] == lax.broadcasted_iota(jnp.int32, (NUM_GRAPHS, NBLK), 0))
    acc_ref[...] += jnp.dot(oh.astype(jnp.float32), h_ref[...],
                            preferred_element_type=jnp.float32)

    @pl.when(i == N // NBLK - 1)
    def _():
        t = _silu(jnp.dot(acc_ref[...], wo1_ref[...],
                          preferred_element_type=jnp.float32) + bo1_ref[...])
        y = jnp.sum(t * wo2t_ref[...], axis=1, keepdims=True) + cst_ref[0, 0]
        out_ref[...] = jnp.broadcast_to(y, (NUM_GRAPHS, H))


@jax.jit
def _tc_pool(b3, h, wo1, bo1, wo2t, cst):
    return pl.pallas_call(
        _pool_body,
        grid=(N // NBLK,),
        in_specs=[
            pl.BlockSpec((1, 1, NBLK), lambda i: (i, 0, 0)),
            pl.BlockSpec((NBLK, H), lambda i: (i, 0)),
            pl.BlockSpec((H, H), lambda i: (0, 0)),
            pl.BlockSpec((1, H), lambda i: (0, 0)),
            pl.BlockSpec((1, H), lambda i: (0, 0)),
            pl.BlockSpec((8, 128), lambda i: (0, 0)),
        ],
        out_specs=pl.BlockSpec((NUM_GRAPHS, H), lambda i: (0, 0)),
        out_shape=jax.ShapeDtypeStruct((NUM_GRAPHS, H), jnp.float32),
        scratch_shapes=[pltpu.VMEM((NUM_GRAPHS, H), jnp.float32)],
    )(b3, h, wo1, bo1, wo2t, cst)


def kernel(z, pos, edge_index, batch, params):
    z3 = z.astype(jnp.int32).reshape(N // NBLK, 1, NBLK)
    b3 = batch.astype(jnp.int32).reshape(N // NBLK, 1, NBLK)
    src_w = edge_index[0].astype(jnp.int32).reshape(NW, NCH, CH)
    dst_w = edge_index[1].astype(jnp.int32).reshape(NW, NCH, CH)
    dst_s = edge_index[1].astype(jnp.int32).reshape(NS, NCHS, CH)
    dst_f = edge_index[1].astype(jnp.int32)

    pos0 = pos[:, 2, :]
    px, py, pz = pos0[:, 0], pos0[:, 1], pos0[:, 2]

    emb_p = jnp.zeros((H, H), jnp.float32).at[:100, :].set(params["embed"])
    h = _tc_embed(z3, emb_p)

    for layer in params["layers"]:
        w1 = layer["edge1"]["W"]
        ts, td = _tc_tables(h, w1[:H], w1[H:2 * H],
                            layer["edge1"]["b"].reshape(1, H))
        gs, gd, rx, ry, rz = _sc_gather(ts, td, src_w, dst_w, px, py, pz)
        cst_e = jnp.zeros((8, 128), jnp.float32).at[0, 0].set(
            layer["coord"]["b"][0])
        m, rwx, rwy, rwz = _tc_edge(
            gs, gd,
            rx.reshape(E // EBLK, 1, EBLK),
            ry.reshape(E // EBLK, 1, EBLK),
            rz.reshape(E // EBLK, 1, EBLK),
            w1[2 * H].reshape(1, H), layer["edge2"]["W"],
            layer["edge2"]["b"].reshape(1, H),
            layer["coord"]["W"].reshape(1, H), cst_e)
        agg, ax, ay, az, ac = _sc_scatter(
            m, rwx.reshape(E), rwy.reshape(E), rwz.reshape(E), dst_s, dst_f)
        wn1 = layer["node1"]["W"]
        h, px3, py3, pz3 = _tc_node(
            agg, ax, ay, az, ac,
            px.reshape(N // NBLK, 1, NBLK),
            py.reshape(N // NBLK, 1, NBLK),
            pz.reshape(N // NBLK, 1, NBLK),
            h, wn1[:H], wn1[H:], layer["node1"]["b"].reshape(1, H),
            layer["node2"]["W"], layer["node2"]["b"].reshape(1, H))
        px, py, pz = px3.reshape(N), py3.reshape(N), pz3.reshape(N)

    cst_o = jnp.zeros((8, 128), jnp.float32).at[0, 0].set(
        params["out2"]["b"][0])
    out = _tc_pool(b3, h, params["out1"]["W"],
                   params["out1"]["b"].reshape(1, H),
                   params["out2"]["W"].reshape(1, H), cst_o)
    return out[:, :1]


# speedup vs baseline: 3.4481x; 1.1718x over previous
import dataclasses
import functools

import jax
import jax.numpy as jnp
from jax import lax
from jax.experimental import pallas as pl
from jax.experimental.pallas import tpu as pltpu
from jax.experimental.pallas import tpu_sc as plsc

N = 10000
E = 320000
H = 128
NUM_GRAPHS = 256

NC, NS = 2, 16
NW = NC * NS
EPW = E // NW
CH = 80
NCH = EPW // CH
NG = CH // 16
EPS = E // NS
NCHS = EPS // CH
HALF = N // 2
ACCR = HALF + 8
EPQ = E // 4
SB = 4000
NSB = EPQ // SB
ZR = 16

NBLK = 1000
EBLK = 1000


@functools.cache
def _sc_mesh():
    return plsc.VectorSubcoreMesh(core_axis_name="c", subcore_axis_name="s")


@functools.cache
def _sc_params():
    cp = pltpu.CompilerParams()
    if "needs_layout_passes" in pltpu.CompilerParams.__dataclass_fields__:
        cp = dataclasses.replace(cp, needs_layout_passes=False)
    return cp


@jax.jit
def _sc_gather(tsrc, tdst, src_w, dst_w, px, py, pz):
    @functools.partial(
        pl.kernel,
        out_type=(
            jax.ShapeDtypeStruct((E, H), jnp.float32),
            jax.ShapeDtypeStruct((E, H), jnp.float32),
            jax.ShapeDtypeStruct((E,), jnp.float32),
            jax.ShapeDtypeStruct((E,), jnp.float32),
            jax.ShapeDtypeStruct((E,), jnp.float32),
        ),
        mesh=_sc_mesh(),
        compiler_params=_sc_params(),
        scratch_types=[
            pltpu.VMEM((NCH, CH), jnp.int32),
            pltpu.VMEM((NCH, CH), jnp.int32),
            pltpu.VMEM((CH, H), jnp.float32),
            pltpu.VMEM((CH, H), jnp.float32),
            pltpu.VMEM((CH, H), jnp.float32),
            pltpu.VMEM((CH, H), jnp.float32),
            pltpu.VMEM((N,), jnp.float32),
            pltpu.VMEM((N,), jnp.float32),
            pltpu.VMEM((N,), jnp.float32),
            pltpu.VMEM((2, CH), jnp.float32),
            pltpu.VMEM((2, CH), jnp.float32),
            pltpu.VMEM((2, CH), jnp.float32),
            pltpu.SemaphoreType.DMA,
            pltpu.SemaphoreType.DMA,
            pltpu.SemaphoreType.DMA,
            pltpu.SemaphoreType.DMA,
            pltpu.SemaphoreType.DMA,
            pltpu.SemaphoreType.DMA,
        ],
    )
    def k(tsrc_hbm, tdst_hbm, src_hbm, dst_hbm, px_hbm, py_hbm, pz_hbm,
          gs_hbm, gd_hbm, rx_hbm, ry_hbm, rz_hbm,
          isv, idv, bs0, bd0, bs1, bd1, pxv, pyv, pzv, rsx, rsy, rsz,
          sem_g0, sem_g1, sem_w0, sem_w1, sem_r0, sem_r1):
        wid = lax.axis_index("s") * NC + lax.axis_index("c")
        pltpu.sync_copy(src_hbm.at[wid], isv)
        pltpu.sync_copy(dst_hbm.at[wid], idv)
        pltpu.sync_copy(px_hbm, pxv)
        pltpu.sync_copy(py_hbm, pyv)
        pltpu.sync_copy(pz_hbm, pzv)
        base = wid * EPW

        bufs = ((bs0, bd0, sem_g0, sem_w0), (bs1, bd1, sem_g1, sem_w1))

        def issue_gather(j, b):
            bs, bd, sem_g, _ = bufs[b]
            pltpu.async_copy(tsrc_hbm.at[isv.at[j]], bs, sem_g)
            pltpu.async_copy(tdst_hbm.at[idv.at[j]], bd, sem_g)

        def wait_gather(b):
            bs, bd, sem_g, _ = bufs[b]
            pltpu.make_async_copy(tsrc_hbm.at[pl.ds(0, CH)], bs, sem_g).wait()
            pltpu.make_async_copy(tdst_hbm.at[pl.ds(0, CH)], bd, sem_g).wait()

        def issue_write(j, b):
            bs, bd, _, sem_w = bufs[b]
            o2 = pl.ds(base + j * CH, CH)
            pltpu.async_copy(bs, gs_hbm.at[o2], sem_w)
            pltpu.async_copy(bd, gd_hbm.at[o2], sem_w)

        def wait_write(b):
            bs, bd, _, sem_w = bufs[b]
            pltpu.make_async_copy(bs, gs_hbm.at[pl.ds(0, CH)], sem_w).wait()
            pltpu.make_async_copy(bd, gd_hbm.at[pl.ds(0, CH)], sem_w).wait()

        sem_rs = (sem_r0, sem_r1)

        def wait_rel(b):
            @pl.loop(0, 3)
            def _(_q):
                pltpu.make_async_copy(
                    rsx.at[0], rx_hbm.at[pl.ds(0, CH)], sem_rs[b]).wait()

        def rel(j, b):
            @pl.loop(0, NG)
            def _(g):
                is16 = isv.at[j][pl.ds(g * 16, 16)]
                id16 = idv.at[j][pl.ds(g * 16, 16)]
                o = pl.ds(g * 16, 16)
                rsx.at[b, o][...] = (plsc.load_gather(pxv, [is16])
                                     - plsc.load_gather(pxv, [id16]))
                rsy.at[b, o][...] = (plsc.load_gather(pyv, [is16])
                                     - plsc.load_gather(pyv, [id16]))
                rsz.at[b, o][...] = (plsc.load_gather(pzv, [is16])
                                     - plsc.load_gather(pzv, [id16]))
            o2 = pl.ds(base + j * CH, CH)
            pltpu.sync_copy(rsx.at[b], rx_hbm.at[o2])
            pltpu.sync_copy(rsy.at[b], ry_hbm.at[o2])
            pltpu.sync_copy(rsz.at[b], rz_hbm.at[o2])

        issue_gather(0, 0)

        NPAIR = (NCH + 1) // 2

        @pl.loop(0, NPAIR)
        def _(jj):
            for b in (0, 1):
                j = jj * 2 + b
                bn = 1 - b

                nxt = j + 1

                @pl.when(nxt < NCH)
                def _():
                    @pl.when(nxt >= 2)
                    def _():
                        wait_write(bn)

                    issue_gather(nxt, bn)

                @pl.when(j < NCH)
                def _():
                    rel(j, b)
                    wait_gather(b)
                    issue_write(j, b)

        wait_write(0)
        wait_write(1)

    return k(tsrc, tdst, src_w, dst_w, px, py, pz)


@jax.jit
def _sc_scatter(m, rwx, rwy, rwz, dst_s, dst_f):
    @functools.partial(
        pl.kernel,
        out_type=(
            jax.ShapeDtypeStruct((N, H), jnp.float32),
            jax.ShapeDtypeStruct((N // NBLK, 4, 1, NBLK), jnp.float32),
            jax.ShapeDtypeStruct((N // NBLK, 4, 1, NBLK), jnp.float32),
            jax.ShapeDtypeStruct((N // NBLK, 4, 1, NBLK), jnp.float32),
            jax.ShapeDtypeStruct((N // NBLK, 4, 1, NBLK), jnp.float32),
        ),
        mesh=_sc_mesh(),
        compiler_params=_sc_params(),
        scratch_types=[
            pltpu.VMEM((NCHS, CH), jnp.int32),
            pltpu.VMEM((CH, H), jnp.float32),
            pltpu.VMEM((CH, H), jnp.float32),
            pltpu.VMEM((ZR, H), jnp.float32),
            pltpu.VMEM((SB,), jnp.int32),
            pltpu.VMEM((SB,), jnp.float32),
            pltpu.VMEM((N,), jnp.float32),
            pltpu.VMEM((N,), jnp.int32),
            pltpu.VMEM((1, CH), jnp.int32),
            pltpu.VMEM_SHARED((ACCR, H), jnp.float32),
            pltpu.SemaphoreType.DMA,
            pltpu.SemaphoreType.DMA,
            pltpu.SemaphoreType.DMA,
        ],
    )
    def k(m_hbm, rwx_hbm, rwy_hbm, rwz_hbm, dst_hbm, dstf_hbm,
          agg_hbm, ax_hbm, ay_hbm, az_hbm, ac_hbm,
          idxv, mbuf0, mbuf1, zbuf, idxb, vb, acc1, markv, idm1, acc_sh,
          sem, sem_m0, sem_m1):
        cid = lax.axis_index("c")
        sid = lax.axis_index("s")

        z16 = jnp.zeros((16,), jnp.float32)

        @pl.loop(0, ZR)
        def _(r):
            @pl.loop(0, H // 16)
            def _(c):
                zbuf.at[r, pl.ds(c * 16, 16)][...] = z16

        zlo = sid * 320

        @pl.loop(0, 20)
        def _(r):
            @pl.when((zlo + r * ZR) < ACCR)
            def _():
                pltpu.async_copy(zbuf, acc_sh.at[pl.ds(zlo + r * ZR, ZR)], sem)

        @pl.loop(0, 20)
        def _(r):
            @pl.when((zlo + r * ZR) < ACCR)
            def _():
                pltpu.make_async_copy(
                    zbuf, acc_sh.at[pl.ds(zlo, ZR)], sem).wait()

        @pl.loop(0, N // 16)
        def _(r):
            acc1.at[pl.ds(r * 16, 16)][...] = z16

        pltpu.sync_copy(dst_hbm.at[sid], idxv)

        lo = cid * HALF

        @pl.loop(0, NCHS)
        def _(j):
            @pl.loop(0, NG)
            def _(g):
                o = pl.ds(g * 16, 16)
                v = idxv.at[j][o]
                inr = (v >= lo) & (v < lo + HALF)
                idxv.at[j][o] = jnp.where(inr, v - lo, HALF)

        plsc.subcore_barrier()

        base = sid * EPS

        iota16 = lax.iota(jnp.int32, 16)
        mbufs = (mbuf0, mbuf1)

        sem_ms = (sem_m0, sem_m1)

        def issue_m(j, b):
            pltpu.async_copy(
                m_hbm.at[pl.ds(base + j * CH, CH)], mbufs[b], sem_ms[b])

        def wait_m(b):
            pltpu.make_async_copy(
                m_hbm.at[pl.ds(0, CH)], mbufs[b], sem_ms[b]).wait()

        def process(j, b):
            mbuf = mbufs[b]
            rem0 = []
            for g in range(NG):
                vg = idxv.at[j][pl.ds(g * 16, 16)]
                rem0.append(vg != HALF)

            def cond(carry):
                return carry[NG] > 0

            def body(carry):
                rem = carry[:NG]
                slot = []
                for g in range(NG):
                    vg = idxv.at[j][pl.ds(g * 16, 16)]
                    sg = j * 128 + g * 16 + iota16
                    slot.append((vg, sg))
                    plsc.store_scatter(markv, [vg], sg, mask=rem[g])
                newrem = []
                total = jnp.zeros((), jnp.int32)
                for g in range(NG):
                    vg, sg = slot[g]
                    rb = plsc.load_gather(markv, [vg])
                    own = rem[g] & (rb == sg)
                    idm1.at[0][pl.ds(g * 16, 16)] = jnp.where(own, vg, HALF)
                    nr = rem[g] & jnp.logical_not(own)
                    newrem.append(nr)
                    total = total + jnp.sum(nr.astype(jnp.int32))
                pltpu.sync_copy(mbuf, acc_sh.at[idm1.at[0]], add=True)
                return tuple(newrem) + (total,)

            n0 = jnp.zeros((), jnp.int32)
            for g in range(NG):
                n0 = n0 + jnp.sum(rem0[g].astype(jnp.int32))
            lax.while_loop(cond, body, tuple(rem0) + (n0,))

        issue_m(0, 0)

        @pl.loop(0, NCHS // 2)
        def _(jj):
            for b in (0, 1):
                j = jj * 2 + b

                @pl.when(j + 1 < NCHS)
                def _():
                    issue_m(j + 1, 1 - b)

                wait_m(b)
                process(j, b)

        one16 = jnp.ones((16,), jnp.float32)
        combo = cid * 8 + sid
        comp = combo % 4
        quarter = combo // 4

        @pl.when(sid < 8)
        def _():
            @pl.loop(0, NSB)
            def _(b):
                qbase = quarter * EPQ + b * SB
                pltpu.sync_copy(dstf_hbm.at[pl.ds(qbase, SB)], idxb)

                @pl.when(comp == 0)
                def _():
                    pltpu.sync_copy(rwx_hbm.at[pl.ds(qbase, SB)], vb)

                @pl.when(comp == 1)
                def _():
                    pltpu.sync_copy(rwy_hbm.at[pl.ds(qbase, SB)], vb)

                @pl.when(comp == 2)
                def _():
                    pltpu.sync_copy(rwz_hbm.at[pl.ds(qbase, SB)], vb)

                @pl.loop(0, SB // 16)
                def _(g):
                    o = pl.ds(g * 16, 16)
                    id16 = idxb.at[o][...]

                    @pl.when(comp < 3)
                    def _():
                        plsc.addupdate_scatter(acc1, [id16], vb.at[o][...])

                    @pl.when(comp == 3)
                    def _():
                        plsc.addupdate_scatter(acc1, [id16], one16)

        plsc.subcore_barrier()

        dlo = sid * 320

        @pl.when(sid < NS - 1)
        def _():
            pltpu.sync_copy(acc_sh.at[pl.ds(dlo, 320)],
                            agg_hbm.at[pl.ds(cid * HALF + dlo, 320)])

        @pl.when(sid == NS - 1)
        def _():
            pltpu.sync_copy(acc_sh.at[pl.ds(dlo, 200)],
                            agg_hbm.at[pl.ds(cid * HALF + dlo, 200)])

        @pl.when(sid < 8)
        def _():
            @pl.loop(0, N // NBLK)
            def _(t):
                o = pl.ds(t * NBLK, NBLK)

                @pl.when(comp == 0)
                def _():
                    pltpu.sync_copy(acc1.at[o], ax_hbm.at[t, quarter, 0])

                @pl.when(comp == 1)
                def _():
                    pltpu.sync_copy(acc1.at[o], ay_hbm.at[t, quarter, 0])

                @pl.when(comp == 2)
                def _():
                    pltpu.sync_copy(acc1.at[o], az_hbm.at[t, quarter, 0])

                @pl.when(comp == 3)
                def _():
                    pltpu.sync_copy(acc1.at[o], ac_hbm.at[t, quarter, 0])

    return k(m, rwx, rwy, rwz, dst_s, dst_f)


def _silu(x):
    return x * (1.0 / (1.0 + jnp.exp(-x)))


def _embed_body(z_ref, emb_ref, h_ref):
    zb = z_ref[0, 0, :]
    oh = (zb[:, None] == lax.broadcasted_iota(jnp.int32, (NBLK, H), 1))
    h_ref[...] = jnp.dot(oh.astype(jnp.float32), emb_ref[...],
                         preferred_element_type=jnp.float32)


@jax.jit
def _tc_embed(z3, emb_p):
    return pl.pallas_call(
        _embed_body,
        grid=(N // NBLK,),
        in_specs=[
            pl.BlockSpec((1, 1, NBLK), lambda i: (i, 0, 0)),
            pl.BlockSpec((H, H), lambda i: (0, 0)),
        ],
        out_specs=pl.BlockSpec((NBLK, H), lambda i: (i, 0)),
        out_shape=jax.ShapeDtypeStruct((N, H), jnp.float32),
    )(z3, emb_p)


def _tables_body(h_ref, wa_ref, wb_ref, b1_ref, ts_ref, td_ref):
    h = h_ref[...]
    ts_ref[...] = jnp.dot(h, wa_ref[...], preferred_element_type=jnp.float32)
    td_ref[...] = (jnp.dot(h, wb_ref[...], preferred_element_type=jnp.float32)
                   + b1_ref[...])


@jax.jit
def _tc_tables(h, wa, wb, b1):
    return pl.pallas_call(
        _tables_body,
        grid=(N // NBLK,),
        in_specs=[
            pl.BlockSpec((NBLK, H), lambda i: (i, 0)),
            pl.BlockSpec((H, H), lambda i: (0, 0)),
            pl.BlockSpec((H, H), lambda i: (0, 0)),
            pl.BlockSpec((1, H), lambda i: (0, 0)),
        ],
        out_specs=(
            pl.BlockSpec((NBLK, H), lambda i: (i, 0)),
            pl.BlockSpec((NBLK, H), lambda i: (i, 0)),
        ),
        out_shape=(
            jax.ShapeDtypeStruct((N, H), jnp.float32),
            jax.ShapeDtypeStruct((N, H), jnp.float32),
        ),
    )(h, wa, wb, b1)


def _edge_body(gs_ref, gd_ref, rx_ref, ry_ref, rz_ref,
               wd_ref, w2_ref, b2_ref, wct_ref, cst_ref,
               m_ref, ox_ref, oy_ref, oz_ref):
    rx = rx_ref[0, 0, :]
    ry = ry_ref[0, 0, :]
    rz = rz_ref[0, 0, :]
    dist2 = (rx * rx + ry * ry + rz * rz)[:, None]
    t = gs_ref[...] + gd_ref[...] + dist2 * wd_ref[...]
    m1 = _silu(t)
    m = _silu(jnp.dot(m1, w2_ref[...], preferred_element_type=jnp.float32)
              + b2_ref[...])
    wc = jnp.sum(m * wct_ref[...], axis=1) + cst_ref[0, 0]
    m_ref[...] = m
    ox_ref[0, 0, :] = rx * wc
    oy_ref[0, 0, :] = ry * wc
    oz_ref[0, 0, :] = rz * wc


@jax.jit
def _tc_edge(gs, gd, rx3, ry3, rz3, wd, w2, b2, wct, cst):
    v3 = pl.BlockSpec((1, 1, EBLK), lambda i: (i, 0, 0))
    w128 = pl.BlockSpec((1, H), lambda i: (0, 0))
    return pl.pallas_call(
        _edge_body,
        grid=(E // EBLK,),
        in_specs=[
            pl.BlockSpec((EBLK, H), lambda i: (i, 0)),
            pl.BlockSpec((EBLK, H), lambda i: (i, 0)),
            v3, v3, v3,
            w128,
            pl.BlockSpec((H, H), lambda i: (0, 0)),
            w128, w128,
            pl.BlockSpec((8, 128), lambda i: (0, 0)),
        ],
        out_specs=(
            pl.BlockSpec((EBLK, H), lambda i: (i, 0)),
            v3, v3, v3,
        ),
        out_shape=(
            jax.ShapeDtypeStruct((E, H), jnp.float32),
            jax.ShapeDtypeStruct((E // EBLK, 1, EBLK), jnp.float32),
            jax.ShapeDtypeStruct((E // EBLK, 1, EBLK), jnp.float32),
            jax.ShapeDtypeStruct((E // EBLK, 1, EBLK), jnp.float32),
        ),
    )(gs, gd, rx3, ry3, rz3, wd, w2, b2, wct, cst)


def _node_body(agg_ref, ax_ref, ay_ref, az_ref, ac_ref,
               px_ref, py_ref, pz_ref, h_ref,
               wna_ref, wnb_ref, bn1_ref, wn2_ref, bn2_ref,
               h_out, px_out, py_out, pz_out):
    agg = agg_ref[...]
    deg = jnp.sum(ac_ref[0, :, 0, :], axis=0) + 1.0
    px_out[0, 0, :] = px_ref[0, 0, :] + jnp.sum(ax_ref[0, :, 0, :], axis=0) / deg
    py_out[0, 0, :] = py_ref[0, 0, :] + jnp.sum(ay_ref[0, :, 0, :], axis=0) / deg
    pz_out[0, 0, :] = pz_ref[0, 0, :] + jnp.sum(az_ref[0, :, 0, :], axis=0) / deg
    h = h_ref[...]
    t = _silu(jnp.dot(h, wna_ref[...], preferred_element_type=jnp.float32)
              + jnp.dot(agg, wnb_ref[...], preferred_element_type=jnp.float32)
              + bn1_ref[...])
    h_out[...] = h + jnp.dot(t, wn2_ref[...],
                             preferred_element_type=jnp.float32) + bn2_ref[...]


@jax.jit
def _tc_node(agg, ax, ay, az, ac, px3, py3, pz3, h,
             wna, wnb, bn1, wn2, bn2):
    v3 = pl.BlockSpec((1, 1, NBLK), lambda i: (i, 0, 0))
    part = pl.BlockSpec((1, 4, 1, NBLK), lambda i: (i, 0, 0, 0))
    return pl.pallas_call(
        _node_body,
        grid=(N // NBLK,),
        in_specs=[
            pl.BlockSpec((NBLK, H), lambda i: (i, 0)),
            part, part, part, part,
            v3, v3, v3,
            pl.BlockSpec((NBLK, H), lambda i: (i, 0)),
            pl.BlockSpec((H, H), lambda i: (0, 0)),
            pl.BlockSpec((H, H), lambda i: (0, 0)),
            pl.BlockSpec((1, H), lambda i: (0, 0)),
            pl.BlockSpec((H, H), lambda i: (0, 0)),
            pl.BlockSpec((1, H), lambda i: (0, 0)),
        ],
        out_specs=(
            pl.BlockSpec((NBLK, H), lambda i: (i, 0)),
            v3, v3, v3,
        ),
        out_shape=(
            jax.ShapeDtypeStruct((N, H), jnp.float32),
            jax.ShapeDtypeStruct((N // NBLK, 1, NBLK), jnp.float32),
            jax.ShapeDtypeStruct((N // NBLK, 1, NBLK), jnp.float32),
            jax.ShapeDtypeStruct((N // NBLK, 1, NBLK), jnp.float32),
        ),
    )(agg, ax, ay, az, ac, px3, py3, pz3, h, wna, wnb, bn1, wn2, bn2)


def _pool_body(b_ref, h_ref, wo1_ref, bo1_ref, wo2t_ref, cst_ref, out_ref,
               acc_ref):
    i = pl.program_id(0)

    @pl.when(i == 0)
    def _():
        acc_ref[...] = jnp.zeros((NUM_GRAPHS, H), jnp.float32)

    bb = b_ref[0, 0, :]
    oh = (bb[None, :] == lax.broadcasted_iota(jnp.int32, (NUM_GRAPHS, NBLK), 0))
    acc_ref[...] += jnp.dot(oh.astype(jnp.float32), h_ref[...],
                            preferred_element_type=jnp.float32)

    @pl.when(i == N // NBLK - 1)
    def _():
        t = _silu(jnp.dot(acc_ref[...], wo1_ref[...],
                          preferred_element_type=jnp.float32) + bo1_ref[...])
        y = jnp.sum(t * wo2t_ref[...], axis=1, keepdims=True) + cst_ref[0, 0]
        out_ref[...] = jnp.broadcast_to(y, (NUM_GRAPHS, H))


@jax.jit
def _tc_pool(b3, h, wo1, bo1, wo2t, cst):
    return pl.pallas_call(
        _pool_body,
        grid=(N // NBLK,),
        in_specs=[
            pl.BlockSpec((1, 1, NBLK), lambda i: (i, 0, 0)),
            pl.BlockSpec((NBLK, H), lambda i: (i, 0)),
            pl.BlockSpec((H, H), lambda i: (0, 0)),
            pl.BlockSpec((1, H), lambda i: (0, 0)),
            pl.BlockSpec((1, H), lambda i: (0, 0)),
            pl.BlockSpec((8, 128), lambda i: (0, 0)),
        ],
        out_specs=pl.BlockSpec((NUM_GRAPHS, H), lambda i: (0, 0)),
        out_shape=jax.ShapeDtypeStruct((NUM_GRAPHS, H), jnp.float32),
        scratch_shapes=[pltpu.VMEM((NUM_GRAPHS, H), jnp.float32)],
    )(b3, h, wo1, bo1, wo2t, cst)


def kernel(z, pos, edge_index, batch, params):
    z3 = z.astype(jnp.int32).reshape(N // NBLK, 1, NBLK)
    b3 = batch.astype(jnp.int32).reshape(N // NBLK, 1, NBLK)
    src_w = edge_index[0].astype(jnp.int32).reshape(NW, NCH, CH)
    dst_w = edge_index[1].astype(jnp.int32).reshape(NW, NCH, CH)
    dst_s = edge_index[1].astype(jnp.int32).reshape(NS, NCHS, CH)
    dst_f = edge_index[1].astype(jnp.int32)

    pos0 = pos[:, 2, :]
    px, py, pz = pos0[:, 0], pos0[:, 1], pos0[:, 2]

    emb_p = jnp.zeros((H, H), jnp.float32).at[:100, :].set(params["embed"])
    h = _tc_embed(z3, emb_p)

    for layer in params["layers"]:
        w1 = layer["edge1"]["W"]
        ts, td = _tc_tables(h, w1[:H], w1[H:2 * H],
                            layer["edge1"]["b"].reshape(1, H))
        gs, gd, rx, ry, rz = _sc_gather(ts, td, src_w, dst_w, px, py, pz)
        cst_e = jnp.zeros((8, 128), jnp.float32).at[0, 0].set(
            layer["coord"]["b"][0])
        m, rwx, rwy, rwz = _tc_edge(
            gs, gd,
            rx.reshape(E // EBLK, 1, EBLK),
            ry.reshape(E // EBLK, 1, EBLK),
            rz.reshape(E // EBLK, 1, EBLK),
            w1[2 * H].reshape(1, H), layer["edge2"]["W"],
            layer["edge2"]["b"].reshape(1, H),
            layer["coord"]["W"].reshape(1, H), cst_e)
        agg, ax, ay, az, ac = _sc_scatter(
            m, rwx.reshape(E), rwy.reshape(E), rwz.reshape(E), dst_s, dst_f)
        wn1 = layer["node1"]["W"]
        h, px3, py3, pz3 = _tc_node(
            agg, ax, ay, az, ac,
            px.reshape(N // NBLK, 1, NBLK),
            py.reshape(N // NBLK, 1, NBLK),
            pz.reshape(N // NBLK, 1, NBLK),
            h, wn1[:H], wn1[H:], layer["node1"]["b"].reshape(1, H),
            layer["node2"]["W"], layer["node2"]["b"].reshape(1, H))
        px, py, pz = px3.reshape(N), py3.reshape(N), pz3.reshape(N)

    cst_o = jnp.zeros((8, 128), jnp.float32).at[0, 0].set(
        params["out2"]["b"][0])
    out = _tc_pool(b3, h, params["out1"]["W"],
                   params["out1"]["b"].reshape(1, H),
                   params["out2"]["W"].reshape(1, H), cst_o)
    return out[:, :1]


# speedup vs baseline: 4.8700x; 1.4124x over previous
import dataclasses
import functools

import jax
import jax.numpy as jnp
from jax import lax
from jax.experimental import pallas as pl
from jax.experimental.pallas import tpu as pltpu
from jax.experimental.pallas import tpu_sc as plsc

N = 10000
E = 320000
H = 128
NUM_GRAPHS = 256

EH0 = 163840
EH1 = E - EH0

NC, NS = 2, 16
NW = NC * NS
CH = 80
NG = CH // 16
HALF = N // 2
ACCR = HALF + 8
ZR = 16

NBLK = 1000
EBLK = 1280


@functools.cache
def _sc_mesh():
    return plsc.VectorSubcoreMesh(core_axis_name="c", subcore_axis_name="s")


@functools.cache
def _sc_params():
    cp = pltpu.CompilerParams()
    if "needs_layout_passes" in pltpu.CompilerParams.__dataclass_fields__:
        cp = dataclasses.replace(cp, needs_layout_passes=False)
    return cp


@functools.cache
def _build_gather(eh):
    epw = eh // NW
    nch = epw // CH
    npair = (nch + 1) // 2

    @jax.jit
    def gather(tsrc, tdst, src_w, dst_w, px, py, pz):
        @functools.partial(
            pl.kernel,
            out_type=(
                jax.ShapeDtypeStruct((eh, H), jnp.float32),
                jax.ShapeDtypeStruct((eh, H), jnp.float32),
                jax.ShapeDtypeStruct((eh,), jnp.float32),
                jax.ShapeDtypeStruct((eh,), jnp.float32),
                jax.ShapeDtypeStruct((eh,), jnp.float32),
            ),
            mesh=_sc_mesh(),
            compiler_params=_sc_params(),
            scratch_types=[
                pltpu.VMEM((nch, CH), jnp.int32),
                pltpu.VMEM((nch, CH), jnp.int32),
                pltpu.VMEM((CH, H), jnp.float32),
                pltpu.VMEM((CH, H), jnp.float32),
                pltpu.VMEM((CH, H), jnp.float32),
                pltpu.VMEM((CH, H), jnp.float32),
                pltpu.VMEM((N,), jnp.float32),
                pltpu.VMEM((N,), jnp.float32),
                pltpu.VMEM((N,), jnp.float32),
                pltpu.VMEM((2, CH), jnp.float32),
                pltpu.VMEM((2, CH), jnp.float32),
                pltpu.VMEM((2, CH), jnp.float32),
                pltpu.SemaphoreType.DMA,
                pltpu.SemaphoreType.DMA,
                pltpu.SemaphoreType.DMA,
                pltpu.SemaphoreType.DMA,
            ],
        )
        def k(tsrc_hbm, tdst_hbm, src_hbm, dst_hbm, px_hbm, py_hbm, pz_hbm,
              gs_hbm, gd_hbm, rx_hbm, ry_hbm, rz_hbm,
              isv, idv, bs0, bd0, bs1, bd1, pxv, pyv, pzv, rsx, rsy, rsz,
              sem_g0, sem_g1, sem_w0, sem_w1):
            wid = lax.axis_index("s") * NC + lax.axis_index("c")
            pltpu.sync_copy(src_hbm.at[wid], isv)
            pltpu.sync_copy(dst_hbm.at[wid], idv)
            pltpu.sync_copy(px_hbm, pxv)
            pltpu.sync_copy(py_hbm, pyv)
            pltpu.sync_copy(pz_hbm, pzv)
            base = wid * epw

            bufs = ((bs0, bd0, sem_g0, sem_w0), (bs1, bd1, sem_g1, sem_w1))

            def issue_gather(j, b):
                bs, bd, sem_g, _ = bufs[b]
                pltpu.async_copy(tsrc_hbm.at[isv.at[j]], bs, sem_g)
                pltpu.async_copy(tdst_hbm.at[idv.at[j]], bd, sem_g)

            def wait_gather(b):
                bs, bd, sem_g, _ = bufs[b]
                pltpu.make_async_copy(
                    tsrc_hbm.at[pl.ds(0, CH)], bs, sem_g).wait()
                pltpu.make_async_copy(
                    tdst_hbm.at[pl.ds(0, CH)], bd, sem_g).wait()

            def issue_write(j, b):
                bs, bd, _, sem_w = bufs[b]
                o2 = pl.ds(base + j * CH, CH)
                pltpu.async_copy(bs, gs_hbm.at[o2], sem_w)
                pltpu.async_copy(bd, gd_hbm.at[o2], sem_w)

            def wait_write(b):
                bs, bd, _, sem_w = bufs[b]
                pltpu.make_async_copy(
                    bs, gs_hbm.at[pl.ds(0, CH)], sem_w).wait()
                pltpu.make_async_copy(
                    bd, gd_hbm.at[pl.ds(0, CH)], sem_w).wait()

            def rel(j, b):
                @pl.loop(0, NG)
                def _(g):
                    is16 = isv.at[j][pl.ds(g * 16, 16)]
                    id16 = idv.at[j][pl.ds(g * 16, 16)]
                    o = pl.ds(g * 16, 16)
                    rsx.at[b, o][...] = (plsc.load_gather(pxv, [is16])
                                         - plsc.load_gather(pxv, [id16]))
                    rsy.at[b, o][...] = (plsc.load_gather(pyv, [is16])
                                         - plsc.load_gather(pyv, [id16]))
                    rsz.at[b, o][...] = (plsc.load_gather(pzv, [is16])
                                         - plsc.load_gather(pzv, [id16]))
                o2 = pl.ds(base + j * CH, CH)
                pltpu.sync_copy(rsx.at[b], rx_hbm.at[o2])
                pltpu.sync_copy(rsy.at[b], ry_hbm.at[o2])
                pltpu.sync_copy(rsz.at[b], rz_hbm.at[o2])

            issue_gather(0, 0)

            @pl.loop(0, npair)
            def _(jj):
                for b in (0, 1):
                    j = jj * 2 + b
                    bn = 1 - b
                    nxt = j + 1

                    @pl.when(nxt < nch)
                    def _():
                        @pl.when(nxt >= 2)
                        def _():
                            wait_write(bn)

                        issue_gather(nxt, bn)

                    @pl.when(j < nch)
                    def _():
                        rel(j, b)
                        wait_gather(b)
                        issue_write(j, b)

            wait_write(0)
            wait_write(1)

        return k(tsrc, tdst, src_w, dst_w, px, py, pz)

    return gather


@functools.cache
def _build_scatter(eh):
    eps = eh // NS
    nchs = eps // CH
    epq = eh // 4
    sb = epq // 10
    nsb = 10
    sg_n = sb // 16

    @jax.jit
    def scatter(m, rwx, rwy, rwz, dst_s, dst_f):
        @functools.partial(
            pl.kernel,
            out_type=(
                jax.ShapeDtypeStruct((N, H), jnp.float32),
                jax.ShapeDtypeStruct((N // NBLK, 4, 1, NBLK), jnp.float32),
                jax.ShapeDtypeStruct((N // NBLK, 4, 1, NBLK), jnp.float32),
                jax.ShapeDtypeStruct((N // NBLK, 4, 1, NBLK), jnp.float32),
                jax.ShapeDtypeStruct((N // NBLK, 4, 1, NBLK), jnp.float32),
            ),
            mesh=_sc_mesh(),
            compiler_params=_sc_params(),
            scratch_types=[
                pltpu.VMEM((nchs, CH), jnp.int32),
                pltpu.VMEM((CH, H), jnp.float32),
                pltpu.VMEM((CH, H), jnp.float32),
                pltpu.VMEM((ZR, H), jnp.float32),
                pltpu.VMEM((sb,), jnp.int32),
                pltpu.VMEM((sb,), jnp.float32),
                pltpu.VMEM((N,), jnp.float32),
                pltpu.VMEM((N,), jnp.int32),
                pltpu.VMEM((1, CH), jnp.int32),
                pltpu.VMEM_SHARED((ACCR, H), jnp.float32),
                pltpu.SemaphoreType.DMA,
                pltpu.SemaphoreType.DMA,
                pltpu.SemaphoreType.DMA,
            ],
        )
        def k(m_hbm, rwx_hbm, rwy_hbm, rwz_hbm, dst_hbm, dstf_hbm,
              agg_hbm, ax_hbm, ay_hbm, az_hbm, ac_hbm,
              idxv, mbuf0, mbuf1, zbuf, idxb, vb, acc1, markv, idm1, acc_sh,
              sem, sem_m0, sem_m1):
            cid = lax.axis_index("c")
            sid = lax.axis_index("s")

            z16 = jnp.zeros((16,), jnp.float32)

            @pl.loop(0, ZR)
            def _(r):
                @pl.loop(0, H // 16)
                def _(c):
                    zbuf.at[r, pl.ds(c * 16, 16)][...] = z16

            zlo = sid * 320

            @pl.loop(0, 20)
            def _(r):
                @pl.when((zlo + r * ZR) < ACCR)
                def _():
                    pltpu.async_copy(
                        zbuf, acc_sh.at[pl.ds(zlo + r * ZR, ZR)], sem)

            @pl.loop(0, 20)
            def _(r):
                @pl.when((zlo + r * ZR) < ACCR)
                def _():
                    pltpu.make_async_copy(
                        zbuf, acc_sh.at[pl.ds(zlo, ZR)], sem).wait()

            @pl.loop(0, N // 16)
            def _(r):
                acc1.at[pl.ds(r * 16, 16)][...] = z16

            pltpu.sync_copy(dst_hbm.at[sid], idxv)

            lo = cid * HALF

            @pl.loop(0, nchs)
            def _(j):
                @pl.loop(0, NG)
                def _(g):
                    o = pl.ds(g * 16, 16)
                    v = idxv.at[j][o]
                    inr = (v >= lo) & (v < lo + HALF)
                    idxv.at[j][o] = jnp.where(inr, v - lo, HALF)

            plsc.subcore_barrier()

            base = sid * eps
            iota16 = lax.iota(jnp.int32, 16)
            mbufs = (mbuf0, mbuf1)
            sem_ms = (sem_m0, sem_m1)

            def issue_m(j, b):
                pltpu.async_copy(
                    m_hbm.at[pl.ds(base + j * CH, CH)], mbufs[b], sem_ms[b])

            def wait_m(b):
                pltpu.make_async_copy(
                    m_hbm.at[pl.ds(0, CH)], mbufs[b], sem_ms[b]).wait()

            def process(j, b):
                mbuf = mbufs[b]
                rem0 = []
                for g in range(NG):
                    vg = idxv.at[j][pl.ds(g * 16, 16)]
                    rem0.append(vg != HALF)

                def cond(carry):
                    return carry[NG] > 0

                def body(carry):
                    rem = carry[:NG]
                    slot = []
                    for g in range(NG):
                        vg = idxv.at[j][pl.ds(g * 16, 16)]
                        sg = j * 128 + g * 16 + iota16
                        slot.append((vg, sg))
                        plsc.store_scatter(markv, [vg], sg, mask=rem[g])
                    newrem = []
                    total = jnp.zeros((), jnp.int32)
                    for g in range(NG):
                        vg, sg = slot[g]
                        rb = plsc.load_gather(markv, [vg])
                        own = rem[g] & (rb == sg)
                        idm1.at[0][pl.ds(g * 16, 16)] = jnp.where(
                            own, vg, HALF)
                        nr = rem[g] & jnp.logical_not(own)
                        newrem.append(nr)
                        total = total + jnp.sum(nr.astype(jnp.int32))
                    pltpu.sync_copy(mbuf, acc_sh.at[idm1.at[0]], add=True)
                    return tuple(newrem) + (total,)

                n0 = jnp.zeros((), jnp.int32)
                for g in range(NG):
                    n0 = n0 + jnp.sum(rem0[g].astype(jnp.int32))
                lax.while_loop(cond, body, tuple(rem0) + (n0,))

            issue_m(0, 0)

            @pl.loop(0, nchs // 2)
            def _(jj):
                for b in (0, 1):
                    j = jj * 2 + b

                    @pl.when(j + 1 < nchs)
                    def _():
                        issue_m(j + 1, 1 - b)

                    wait_m(b)
                    process(j, b)

            one16 = jnp.ones((16,), jnp.float32)
            combo = cid * 8 + sid
            comp = combo % 4
            quarter = combo // 4

            @pl.when(sid < 8)
            def _():
                @pl.loop(0, nsb)
                def _(b):
                    qbase = quarter * epq + b * sb
                    pltpu.sync_copy(dstf_hbm.at[pl.ds(qbase, sb)], idxb)

                    @pl.when(comp == 0)
                    def _():
                        pltpu.sync_copy(rwx_hbm.at[pl.ds(qbase, sb)], vb)

                    @pl.when(comp == 1)
                    def _():
                        pltpu.sync_copy(rwy_hbm.at[pl.ds(qbase, sb)], vb)

                    @pl.when(comp == 2)
                    def _():
                        pltpu.sync_copy(rwz_hbm.at[pl.ds(qbase, sb)], vb)

                    @pl.loop(0, sg_n)
                    def _(g):
                        o = pl.ds(g * 16, 16)
                        id16 = idxb.at[o][...]

                        @pl.when(comp < 3)
                        def _():
                            plsc.addupdate_scatter(
                                acc1, [id16], vb.at[o][...])

                        @pl.when(comp == 3)
                        def _():
                            plsc.addupdate_scatter(acc1, [id16], one16)

            plsc.subcore_barrier()

            dlo = sid * 320

            @pl.when(sid < NS - 1)
            def _():
                pltpu.sync_copy(acc_sh.at[pl.ds(dlo, 320)],
                                agg_hbm.at[pl.ds(cid * HALF + dlo, 320)])

            @pl.when(sid == NS - 1)
            def _():
                pltpu.sync_copy(acc_sh.at[pl.ds(dlo, 200)],
                                agg_hbm.at[pl.ds(cid * HALF + dlo, 200)])

            @pl.when(sid < 8)
            def _():
                @pl.loop(0, N // NBLK)
                def _(t):
                    o = pl.ds(t * NBLK, NBLK)

                    @pl.when(comp == 0)
                    def _():
                        pltpu.sync_copy(acc1.at[o], ax_hbm.at[t, quarter, 0])

                    @pl.when(comp == 1)
                    def _():
                        pltpu.sync_copy(acc1.at[o], ay_hbm.at[t, quarter, 0])

                    @pl.when(comp == 2)
                    def _():
                        pltpu.sync_copy(acc1.at[o], az_hbm.at[t, quarter, 0])

                    @pl.when(comp == 3)
                    def _():
                        pltpu.sync_copy(acc1.at[o], ac_hbm.at[t, quarter, 0])

        return k(m, rwx, rwy, rwz, dst_s, dst_f)

    return scatter


def _silu(x):
    return x * (1.0 / (1.0 + jnp.exp(-x)))


def _embed_body(z_ref, emb_ref, h_ref):
    zb = z_ref[0, 0, :]
    oh = (zb[:, None] == lax.broadcasted_iota(jnp.int32, (NBLK, H), 1))
    h_ref[...] = jnp.dot(oh.astype(jnp.float32), emb_ref[...],
                         preferred_element_type=jnp.float32)


@jax.jit
def _tc_embed(z3, emb_p):
    return pl.pallas_call(
        _embed_body,
        grid=(N // NBLK,),
        in_specs=[
            pl.BlockSpec((1, 1, NBLK), lambda i: (i, 0, 0)),
            pl.BlockSpec((H, H), lambda i: (0, 0)),
        ],
        out_specs=pl.BlockSpec((NBLK, H), lambda i: (i, 0)),
        out_shape=jax.ShapeDtypeStruct((N, H), jnp.float32),
    )(z3, emb_p)


def _tables_body(h_ref, wa_ref, wb_ref, b1_ref, ts_ref, td_ref):
    h = h_ref[...]
    ts_ref[...] = jnp.dot(h, wa_ref[...], preferred_element_type=jnp.float32)
    td_ref[...] = (jnp.dot(h, wb_ref[...], preferred_element_type=jnp.float32)
                   + b1_ref[...])


@jax.jit
def _tc_tables(h, wa, wb, b1):
    return pl.pallas_call(
        _tables_body,
        grid=(N // NBLK,),
        in_specs=[
            pl.BlockSpec((NBLK, H), lambda i: (i, 0)),
            pl.BlockSpec((H, H), lambda i: (0, 0)),
            pl.BlockSpec((H, H), lambda i: (0, 0)),
            pl.BlockSpec((1, H), lambda i: (0, 0)),
        ],
        out_specs=(
            pl.BlockSpec((NBLK, H), lambda i: (i, 0)),
            pl.BlockSpec((NBLK, H), lambda i: (i, 0)),
        ),
        out_shape=(
            jax.ShapeDtypeStruct((N, H), jnp.float32),
            jax.ShapeDtypeStruct((N, H), jnp.float32),
        ),
    )(h, wa, wb, b1)


def _edge_body(gs_ref, gd_ref, rx_ref, ry_ref, rz_ref,
               wd_ref, w2_ref, b2_ref, wct_ref, cst_ref,
               m_ref, ox_ref, oy_ref, oz_ref):
    rx = rx_ref[0, 0, :]
    ry = ry_ref[0, 0, :]
    rz = rz_ref[0, 0, :]
    dist2 = (rx * rx + ry * ry + rz * rz)[:, None]
    t = gs_ref[...] + gd_ref[...] + dist2 * wd_ref[...]
    m1 = _silu(t)
    m = _silu(jnp.dot(m1, w2_ref[...], preferred_element_type=jnp.float32)
              + b2_ref[...])
    wc = jnp.sum(m * wct_ref[...], axis=1) + cst_ref[0, 0]
    m_ref[...] = m
    ox_ref[0, 0, :] = rx * wc
    oy_ref[0, 0, :] = ry * wc
    oz_ref[0, 0, :] = rz * wc


@jax.jit
def _tc_edge(gs, gd, rx, ry, rz, wd, w2, b2, wct, cst):
    eh = gs.shape[0]
    nb = eh // EBLK
    rx = rx.reshape(nb, 1, EBLK)
    ry = ry.reshape(nb, 1, EBLK)
    rz = rz.reshape(nb, 1, EBLK)
    v1 = pl.BlockSpec((1, 1, EBLK), lambda i: (i, 0, 0))
    w128 = pl.BlockSpec((1, H), lambda i: (0, 0))
    out = pl.pallas_call(
        _edge_body,
        grid=(eh // EBLK,),
        in_specs=[
            pl.BlockSpec((EBLK, H), lambda i: (i, 0)),
            pl.BlockSpec((EBLK, H), lambda i: (i, 0)),
            v1, v1, v1,
            w128,
            pl.BlockSpec((H, H), lambda i: (0, 0)),
            w128, w128,
            pl.BlockSpec((8, 128), lambda i: (0, 0)),
        ],
        out_specs=(
            pl.BlockSpec((EBLK, H), lambda i: (i, 0)),
            v1, v1, v1,
        ),
        out_shape=(
            jax.ShapeDtypeStruct((eh, H), jnp.float32),
            jax.ShapeDtypeStruct((nb, 1, EBLK), jnp.float32),
            jax.ShapeDtypeStruct((nb, 1, EBLK), jnp.float32),
            jax.ShapeDtypeStruct((nb, 1, EBLK), jnp.float32),
        ),
    )(gs, gd, rx, ry, rz, wd, w2, b2, wct, cst)
    m, ox, oy, oz = out
    return m, ox.reshape(eh), oy.reshape(eh), oz.reshape(eh)


def _node_body(p0_ref, p1_ref,
               ax0_ref, ay0_ref, az0_ref, ac0_ref,
               ax1_ref, ay1_ref, az1_ref, ac1_ref,
               px_ref, py_ref, pz_ref, h_ref,
               wna_ref, wnb_ref, bn1_ref, wn2_ref, bn2_ref,
               h_out, px_out, py_out, pz_out):
    agg = p0_ref[...] + p1_ref[...]
    deg = (jnp.sum(ac0_ref[0, :, 0, :], axis=0)
           + jnp.sum(ac1_ref[0, :, 0, :], axis=0) + 1.0)
    sx = (jnp.sum(ax0_ref[0, :, 0, :], axis=0)
          + jnp.sum(ax1_ref[0, :, 0, :], axis=0))
    sy = (jnp.sum(ay0_ref[0, :, 0, :], axis=0)
          + jnp.sum(ay1_ref[0, :, 0, :], axis=0))
    sz = (jnp.sum(az0_ref[0, :, 0, :], axis=0)
          + jnp.sum(az1_ref[0, :, 0, :], axis=0))
    px_out[0, 0, :] = px_ref[0, 0, :] + sx / deg
    py_out[0, 0, :] = py_ref[0, 0, :] + sy / deg
    pz_out[0, 0, :] = pz_ref[0, 0, :] + sz / deg
    h = h_ref[...]
    t = _silu(jnp.dot(h, wna_ref[...], preferred_element_type=jnp.float32)
              + jnp.dot(agg, wnb_ref[...], preferred_element_type=jnp.float32)
              + bn1_ref[...])
    h_out[...] = h + jnp.dot(t, wn2_ref[...],
                             preferred_element_type=jnp.float32) + bn2_ref[...]


@jax.jit
def _tc_node(p0, p1, parts0, parts1, px3, py3, pz3, h,
             wna, wnb, bn1, wn2, bn2):
    v3 = pl.BlockSpec((1, 1, NBLK), lambda i: (i, 0, 0))
    part = pl.BlockSpec((1, 4, 1, NBLK), lambda i: (i, 0, 0, 0))
    nb = pl.BlockSpec((NBLK, H), lambda i: (i, 0))
    wf = pl.BlockSpec((H, H), lambda i: (0, 0))
    wb1 = pl.BlockSpec((1, H), lambda i: (0, 0))
    return pl.pallas_call(
        _node_body,
        grid=(N // NBLK,),
        in_specs=[nb, nb,
                  part, part, part, part,
                  part, part, part, part,
                  v3, v3, v3, nb,
                  wf, wf, wb1, wf, wb1],
        out_specs=(nb, v3, v3, v3),
        out_shape=(
            jax.ShapeDtypeStruct((N, H), jnp.float32),
            jax.ShapeDtypeStruct((N // NBLK, 1, NBLK), jnp.float32),
            jax.ShapeDtypeStruct((N // NBLK, 1, NBLK), jnp.float32),
            jax.ShapeDtypeStruct((N // NBLK, 1, NBLK), jnp.float32),
        ),
    )(p0, p1, *parts0, *parts1, px3, py3, pz3, h, wna, wnb, bn1, wn2, bn2)


def _pool_body(b_ref, h_ref, wo1_ref, bo1_ref, wo2t_ref, cst_ref, out_ref,
               acc_ref):
    i = pl.program_id(0)

    @pl.when(i == 0)
    def _():
        acc_ref[...] = jnp.zeros((NUM_GRAPHS, H), jnp.float32)

    bb = b_ref[0, 0, :]
    oh = (bb[None, :] == lax.broadcasted_iota(jnp.int32, (NUM_GRAPHS, NBLK), 0))
    acc_ref[...] += jnp.dot(oh.astype(jnp.float32), h_ref[...],
                            preferred_element_type=jnp.float32)

    @pl.when(i == N // NBLK - 1)
    def _():
        t = _silu(jnp.dot(acc_ref[...], wo1_ref[...],
                          preferred_element_type=jnp.float32) + bo1_ref[...])
        y = jnp.sum(t * wo2t_ref[...], axis=1, keepdims=True) + cst_ref[0, 0]
        out_ref[...] = jnp.broadcast_to(y, (NUM_GRAPHS, H))


@jax.jit
def _tc_pool(b3, h, wo1, bo1, wo2t, cst):
    return pl.pallas_call(
        _pool_body,
        grid=(N // NBLK,),
        in_specs=[
            pl.BlockSpec((1, 1, NBLK), lambda i: (i, 0, 0)),
            pl.BlockSpec((NBLK, H), lambda i: (i, 0)),
            pl.BlockSpec((H, H), lambda i: (0, 0)),
            pl.BlockSpec((1, H), lambda i: (0, 0)),
            pl.BlockSpec((1, H), lambda i: (0, 0)),
            pl.BlockSpec((8, 128), lambda i: (0, 0)),
        ],
        out_specs=pl.BlockSpec((NUM_GRAPHS, H), lambda i: (0, 0)),
        out_shape=jax.ShapeDtypeStruct((NUM_GRAPHS, H), jnp.float32),
        scratch_shapes=[pltpu.VMEM((NUM_GRAPHS, H), jnp.float32)],
    )(b3, h, wo1, bo1, wo2t, cst)


def kernel(z, pos, edge_index, batch, params):
    z3 = z.astype(jnp.int32).reshape(N // NBLK, 1, NBLK)
    b3 = batch.astype(jnp.int32).reshape(N // NBLK, 1, NBLK)

    src = edge_index[0].astype(jnp.int32)
    dst = edge_index[1].astype(jnp.int32)
    halves = []
    off = 0
    for eh in (EH0, EH1):
        epw = eh // NW
        eps = eh // NS
        sl = slice(off, off + eh)
        halves.append({
            "eh": eh,
            "src_w": src[sl].reshape(NW, epw // CH, CH),
            "dst_w": dst[sl].reshape(NW, epw // CH, CH),
            "dst_s": dst[sl].reshape(NS, eps // CH, CH),
            "dst_f": dst[sl],
        })
        off += eh

    pos0 = pos[:, 2, :]
    px, py, pz = pos0[:, 0], pos0[:, 1], pos0[:, 2]

    emb_p = jnp.zeros((H, H), jnp.float32).at[:100, :].set(params["embed"])
    h = _tc_embed(z3, emb_p)

    for layer in params["layers"]:
        w1 = layer["edge1"]["W"]
        ts, td = _tc_tables(h, w1[:H], w1[H:2 * H],
                            layer["edge1"]["b"].reshape(1, H))
        cst_e = jnp.zeros((8, 128), jnp.float32).at[0, 0].set(
            layer["coord"]["b"][0])

        aggs, parts = [], []
        for hv in halves:
            gs, gd, rx, ry, rz = _build_gather(hv["eh"])(
                ts, td, hv["src_w"], hv["dst_w"], px, py, pz)
            m, rwx, rwy, rwz = _tc_edge(
                gs, gd, rx, ry, rz,
                w1[2 * H].reshape(1, H), layer["edge2"]["W"],
                layer["edge2"]["b"].reshape(1, H),
                layer["coord"]["W"].reshape(1, H), cst_e)
            agg, ax, ay, az, ac = _build_scatter(hv["eh"])(
                m, rwx, rwy, rwz, hv["dst_s"], hv["dst_f"])
            aggs.append(agg)
            parts.append((ax, ay, az, ac))

        wn1 = layer["node1"]["W"]
        h, px3, py3, pz3 = _tc_node(
            aggs[0], aggs[1], parts[0], parts[1],
            px.reshape(N // NBLK, 1, NBLK),
            py.reshape(N // NBLK, 1, NBLK),
            pz.reshape(N // NBLK, 1, NBLK),
            h, wn1[:H], wn1[H:], layer["node1"]["b"].reshape(1, H),
            layer["node2"]["W"], layer["node2"]["b"].reshape(1, H))
        px, py, pz = px3.reshape(N), py3.reshape(N), pz3.reshape(N)

    cst_o = jnp.zeros((8, 128), jnp.float32).at[0, 0].set(
        params["out2"]["b"][0])
    out = _tc_pool(b3, h, params["out1"]["W"],
                   params["out1"]["b"].reshape(1, H),
                   params["out2"]["W"].reshape(1, H), cst_o)
    return out[:, :1]
